# baseline (speedup 1.0000x reference)
import functools
import math

import jax
import jax.numpy as jnp
from jax import lax
from jax.experimental import pallas as pl
from jax.experimental.pallas import tpu as pltpu
from jax.experimental.pallas import tpu_sc as plsc

_N = 1024
_SIZE = 100000
_PAD = 0
_SMOOTH = 0.1
_CONF = 1.0 - _SMOOTH
_EPS = _SMOOTH / (_SIZE - 2)
_CCONST = (_SIZE - 2) * _EPS * math.log(_EPS) + _CONF * math.log(_CONF)

_NTILES = 32
_RPT = _N // _NTILES

_GR = 8
_NGRP = _RPT // _GR
_CCOL = 2560
_NCH = _SIZE // _CCOL
_TAIL0 = _NCH * _CCOL
_TAIL1 = _TAIL0 + 128
_NBUF = 4


def _lane0_f32():
    iota = lax.broadcasted_iota(jnp.int32, (16,), 0)
    return jnp.maximum(1 - jnp.abs(iota), 0).astype(jnp.float32)


def _sc_body(x_hbm, t_hbm, out_hbm, tv, accbuf, cbufs, tbuf0, tbuf1, gbuf,
             c0buf, accs, csems, gsem):
    c = lax.axis_index("c")
    s = lax.axis_index("s")
    wid = s * 2 + c
    iota = lax.broadcasted_iota(jnp.int32, (16,), 0)
    base = wid * _RPT

    pltpu.sync_copy(t_hbm.at[pl.ds(base, _RPT)], tv)

    ts = [tv[pl.ds((k // 16) * 16, 16)][k % 16] for k in range(_RPT)]
    ws = [jnp.minimum(jnp.abs(t), 1).astype(jnp.float32) for t in ts]

    gdescs = []
    for k in range(_RPT):
        off = (ts[k] // 16) * 16
        d = pltpu.make_async_copy(
            x_hbm.at[base + k, pl.ds(off, 16)], gbuf.at[k], gsem)
        d.start()
        gdescs.append(d)

    nchunks = _NGRP * _NCH
    zero16 = jnp.zeros((16,), jnp.float32)
    for k in range(_RPT):
        accs[k] = zero16

    def _chunk_copy(m):
        g = lax.bitwise_and(m, _NGRP - 1)
        ci = lax.shift_right_logical(m, 2)
        slot = lax.bitwise_and(m, _NBUF - 1)
        return pltpu.make_async_copy(
            x_hbm.at[pl.ds(base + g * _GR, _GR), pl.ds(ci * _CCOL, _CCOL)],
            cbufs.at[slot],
            csems.at[slot],
        )

    for m in range(_NBUF):
        _chunk_copy(jnp.int32(m)).start()

    def _main(m, carry):
        g = lax.bitwise_and(m, _NGRP - 1)
        slot = lax.bitwise_and(m, _NBUF - 1)
        _chunk_copy(m).wait()
        for r in range(_GR):
            @plsc.parallel_loop(0, _CCOL, 16, unroll=8, carry=zero16)
            def _row_acc(j, a):
                return a + cbufs[slot, r, pl.ds(j, 16)]
            plsc.addupdate(accs.at[g * _GR + r], _row_acc)

        @pl.when(m + _NBUF < nchunks)
        def _():
            _chunk_copy(m + _NBUF).start()
        return carry

    lax.fori_loop(0, nchunks, _main, jnp.int32(0))

    total = zero16
    misc = jnp.float32(0.0)
    for g in range(_NGRP):
        rows = x_hbm.at[pl.ds(base + g * _GR, _GR), :]
        pltpu.sync_copy(rows.at[:, pl.ds(_TAIL0, 128)], tbuf0)
        pltpu.sync_copy(rows.at[:, pl.ds(_TAIL1, 32)], tbuf1)
        pltpu.sync_copy(rows.at[:, pl.ds(0, 128)], c0buf)
        for r in range(_GR):
            k = g * _GR + r
            rsum = accs[k]
            for j in range(8):
                rsum = rsum + tbuf0[r, pl.ds(j * 16, 16)]
            for j in range(2):
                rsum = rsum + tbuf1[r, pl.ds(j * 16, 16)]
            total = total + rsum * (ws[k] * (-_EPS))
            x0 = c0buf[r, pl.ds(0, 16)][0]
            misc = misc + ws[k] * (_CCONST + _EPS * x0)

    total = total + misc * _lane0_f32()

    gacc = zero16
    for k in range(_RPT):
        gdescs[k].wait()
    for k in range(_RPT):
        off = (ts[k] // 16) * 16
        ind = jnp.maximum(1 - jnp.abs(iota - (ts[k] - off)), 0) * \
            jnp.minimum(jnp.abs(ts[k]), 1)
        gacc = gacc + gbuf[k, pl.ds(0, 16)] * ind.astype(jnp.float32)

    accbuf[...] = total + gacc * (_EPS - _CONF)
    pltpu.sync_copy(accbuf, out_hbm.at[pl.ds(wid * 16, 16)])


@functools.cache
def _get_sc_call():
    return functools.partial(
        pl.kernel,
        out_type=jax.ShapeDtypeStruct((_NTILES * 16,), jnp.float32),
        mesh=plsc.VectorSubcoreMesh(core_axis_name="c", subcore_axis_name="s"),
        scratch_types=[
            pltpu.VMEM((_RPT,), jnp.int32),
            pltpu.VMEM((16,), jnp.float32),
            pltpu.VMEM((_NBUF, _GR, _CCOL), jnp.float32),
            pltpu.VMEM((_GR, 128), jnp.float32),
            pltpu.VMEM((_GR, 32), jnp.float32),
            pltpu.VMEM((_RPT, 16), jnp.float32),
            pltpu.VMEM((_GR, 128), jnp.float32),
            pltpu.VMEM((_RPT, 16), jnp.float32),
            pltpu.SemaphoreType.DMA((_NBUF,)),
            pltpu.SemaphoreType.DMA,
        ],
    )(_sc_body)


def _combine_body(v_ref, out_ref):
    out_ref[...] = jnp.broadcast_to(jnp.sum(v_ref[...]), (1, 1))


_combine_call = pl.pallas_call(
    _combine_body,
    in_specs=[pl.BlockSpec((4, 128), lambda: (0, 0))],
    out_specs=pl.BlockSpec((1, 1), lambda: (0, 0)),
    out_shape=jax.ShapeDtypeStruct((1, 1), jnp.float32),
)


def kernel(x, target):
    target = target.astype(jnp.int32)
    sc_out = _get_sc_call()(x, target)
    return _combine_call(sc_out.reshape(4, 128))[0, 0]


# speedup vs baseline: 2.3988x; 2.3988x over previous
import functools
import math

import jax
import jax.numpy as jnp
from jax import lax
from jax.experimental import pallas as pl
from jax.experimental.pallas import tpu as pltpu
from jax.experimental.pallas import tpu_sc as plsc

_N = 1024
_SIZE = 100000
_PAD = 0
_SMOOTH = 0.1
_CONF = 1.0 - _SMOOTH
_EPS = _SMOOTH / (_SIZE - 2)
_CCONST = (_SIZE - 2) * _EPS * math.log(_EPS) + _CONF * math.log(_CONF)

_NTILES = 32
_RPT = _N // _NTILES

_BR = 32


def _tc_body(x_ref, t_ref, out_ref):
    i = pl.program_id(0)
    xb = x_ref[...]
    rowsum = jnp.sum(xb, axis=1, keepdims=True)
    per_row = _CCONST + _EPS * (x_ref[:, 0:1] - rowsum)
    valid = t_ref[...] != _PAD
    part = jnp.sum(jnp.where(valid, per_row, 0.0))

    @pl.when(i == 0)
    def _():
        out_ref[...] = jnp.broadcast_to(part, (1, 1))

    @pl.when(i > 0)
    def _():
        out_ref[...] += part


_tc_call = pl.pallas_call(
    _tc_body,
    grid=(_N // _BR,),
    in_specs=[
        pl.BlockSpec((_BR, _SIZE), lambda i: (i, 0)),
        pl.BlockSpec((_BR, 1), lambda i: (i, 0)),
    ],
    out_specs=pl.BlockSpec((1, 1), lambda i: (0, 0)),
    out_shape=jax.ShapeDtypeStruct((1, 1), jnp.float32),
    compiler_params=pltpu.CompilerParams(
        dimension_semantics=("arbitrary",),
    ),
)


def _sc_gather_body(x_hbm, t_hbm, out_hbm, tv, rowbuf, accbuf):
    c = lax.axis_index("c")
    s = lax.axis_index("s")
    wid = s * 2 + c
    base = wid * _RPT
    pltpu.sync_copy(t_hbm.at[pl.ds(base, _RPT)], tv)
    iota = lax.broadcasted_iota(jnp.int32, (16,), 0)
    acc = jnp.zeros((16,), jnp.float32)
    for k in range(_RPT):
        t = tv[pl.ds((k // 16) * 16, 16)][k % 16]
        off = (t // 16) * 16
        pltpu.sync_copy(x_hbm.at[base + k, pl.ds(off, 16)], rowbuf)
        valid = jnp.minimum(jnp.abs(t), 1)
        ind = jnp.maximum(1 - jnp.abs(iota - (t - off)), 0) * valid
        acc = acc + rowbuf[...] * ind.astype(jnp.float32)
    accbuf[...] = acc * (_EPS - _CONF)
    pltpu.sync_copy(accbuf, out_hbm.at[pl.ds(wid * 16, 16)])


@functools.cache
def _get_sc_call():
    return functools.partial(
        pl.kernel,
        out_type=jax.ShapeDtypeStruct((_NTILES * 16,), jnp.float32),
        mesh=plsc.VectorSubcoreMesh(core_axis_name="c", subcore_axis_name="s"),
        scratch_types=[
            pltpu.VMEM((_RPT,), jnp.int32),
            pltpu.VMEM((16,), jnp.float32),
            pltpu.VMEM((16,), jnp.float32),
        ],
    )(_sc_gather_body)


def kernel(x, target):
    target = target.astype(jnp.int32)
    sc_out = _get_sc_call()(x, target)
    tc_out = _tc_call(x, target.reshape(_N, 1))
    return tc_out[0, 0] + jnp.sum(sc_out)
